# Initial kernel scaffold; baseline (speedup 1.0000x reference)
import jax
import jax.numpy as jnp
from jax.experimental import pallas as pl


def kernel(V, E, node_W, edge_W):
    raise NotImplementedError("write your pallas kernel here")


# speedup vs baseline: 2.1337x; 2.1337x over previous
import functools

import jax
import jax.numpy as jnp
from jax import lax
from jax.experimental import pallas as pl
from jax.experimental.pallas import tpu as pltpu
from jax.experimental.pallas import tpu_sc as plsc

NC = 2
NS = 16
NW = NC * NS

L = 16
CHUNK = 80
G = CHUNK // L

NV = 10000
NE = 320000
D = 128

V_TILES = 25
TV = NV // V_TILES
TE = NE // NW


def _body(V_hbm, E_hbm, nW_hbm, eW_hbm, vout_hbm, eout_hbm,
          node_v, edge_v, w2_v, idx_v, out_v):
    cid = lax.axis_index("c")
    sid = lax.axis_index("s")
    wid = sid * NC + cid
    iota = lax.iota(jnp.int32, L)

    pltpu.sync_copy(nW_hbm, node_v)
    pltpu.sync_copy(eW_hbm, edge_v)

    def build_pg(pg, carry):
        pvec = pg * L + iota
        a = pvec >> 4
        b = pvec & 15
        def build_cb(cb, carry2):
            for j in range(L):
                cc = cb * L + j
                cvec = jnp.full((L,), cc, jnp.int32)
                va = plsc.load_gather(edge_v, [a, cvec])
                vb = plsc.load_gather(edge_v, [b, cvec])
                plsc.store_scatter(w2_v, [pvec, cvec], va + vb)
            return carry2
        return lax.fori_loop(0, D // L, build_cb, carry)
    lax.fori_loop(0, 16, build_pg, 0)

    zvec = jnp.zeros((L,), jnp.int32)

    @pl.when(wid < V_TILES)
    def _node_path():
        vbase = wid * TV
        def vchunk(i, carry):
            row0 = vbase + i * CHUNK
            pltpu.sync_copy(V_hbm.at[pl.ds(row0, CHUNK)], idx_v)
            def grp(g, carry2):
                rvec = g * L + iota
                i0 = plsc.load_gather(idx_v, [rvec, zvec])
                i1 = plsc.load_gather(idx_v, [rvec, zvec + 1])
                i2 = plsc.load_gather(idx_v, [rvec, zvec + 2])
                i3 = plsc.load_gather(idx_v, [rvec, zvec + 3])
                def cols(cb, carry3):
                    for j in range(L):
                        cvec = jnp.full((L,), cb * L + j, jnp.int32)
                        acc = (plsc.load_gather(node_v, [i0, cvec])
                               + plsc.load_gather(node_v, [i1, cvec])
                               + plsc.load_gather(node_v, [i2, cvec])
                               + plsc.load_gather(node_v, [i3, cvec]))
                        plsc.store_scatter(out_v, [rvec, cvec], acc)
                    return carry3
                return lax.fori_loop(0, D // L, cols, carry2)
            lax.fori_loop(0, G, grp, 0)
            pltpu.sync_copy(out_v, vout_hbm.at[pl.ds(row0, CHUNK)])
            return carry
        lax.fori_loop(0, TV // CHUNK, vchunk, 0)

    ebase = wid * TE
    def echunk(i, carry):
        row0 = ebase + i * CHUNK
        pltpu.sync_copy(E_hbm.at[pl.ds(row0, CHUNK)], idx_v)
        def grp(g, carry2):
            rvec = g * L + iota
            e0 = plsc.load_gather(idx_v, [rvec, zvec])
            e1 = plsc.load_gather(idx_v, [rvec, zvec + 1])
            e2 = plsc.load_gather(idx_v, [rvec, zvec + 2])
            e3 = plsc.load_gather(idx_v, [rvec, zvec + 3])
            p01 = (e0 << 4) + e1
            p23 = (e2 << 4) + e3
            def cols(cb, carry3):
                for j in range(L):
                    cvec = jnp.full((L,), cb * L + j, jnp.int32)
                    acc = (plsc.load_gather(w2_v, [p01, cvec])
                           + plsc.load_gather(w2_v, [p23, cvec]))
                    plsc.store_scatter(out_v, [rvec, cvec], acc)
                return carry3
            return lax.fori_loop(0, D // L, cols, carry2)
        lax.fori_loop(0, G, grp, 0)
        pltpu.sync_copy(out_v, eout_hbm.at[pl.ds(row0, CHUNK)])
        return carry
    lax.fori_loop(0, TE // CHUNK, echunk, 0)


@jax.jit
def _run(V, E, node_W, edge_W):
    mesh = plsc.VectorSubcoreMesh(core_axis_name="c", subcore_axis_name="s",
                                  num_cores=NC, num_subcores=NS)
    f = pl.kernel(
        _body,
        out_type=(
            jax.ShapeDtypeStruct((NV, D), jnp.float32),
            jax.ShapeDtypeStruct((NE, D), jnp.float32),
        ),
        mesh=mesh,
        scratch_types=[
            pltpu.VMEM((128, D), jnp.float32),
            pltpu.VMEM((16, D), jnp.float32),
            pltpu.VMEM((256, D), jnp.float32),
            pltpu.VMEM((CHUNK, 4), jnp.int32),
            pltpu.VMEM((CHUNK, D), jnp.float32),
        ],
        compiler_params=pltpu.CompilerParams(needs_layout_passes=False),
    )
    return f(V, E, node_W, edge_W)


def kernel(V, E, node_W, edge_W):
    return _run(V, E, node_W, edge_W)


# speedup vs baseline: 3.7358x; 1.7508x over previous
import jax
import jax.numpy as jnp
from jax import lax
from jax.experimental import pallas as pl
from jax.experimental.pallas import tpu as pltpu
from jax.experimental.pallas import tpu_sc as plsc

NC = 2
NS = 16
NW = NC * NS

L = 16
D = 128
NV = 10000
NE = 320000
VPAD = 10240

TE = NE // NW
ECHUNK = 80
EG = ECHUNK // L
ENCHUNKS = TE // ECHUNK
EWORDS = ECHUNK * D

VG_MAX = 20
VROWS_MAX = VG_MAX * L


def _body(Vf_hbm, Ef_hbm, nWf_hbm, eWf_hbm, voutf_hbm, eoutf_hbm,
          node_f, edge_f, w2_f, vidx_f, eidx_f, out_f,
          sem_vin, sem_ein, sem_o0, sem_o1):
    cid = lax.axis_index("c")
    sid = lax.axis_index("s")
    wid = sid * NC + cid
    iota = lax.iota(jnp.int32, L)

    ebase = wid * TE
    vgstart = wid * 19 + jnp.minimum(wid, 17)
    vgn = jnp.where(wid < 17, VG_MAX, VG_MAX - 1)
    vbase = vgstart * L

    vin = pltpu.async_copy(Vf_hbm.at[pl.ds(vbase * 4, VROWS_MAX * 4)],
                           vidx_f, sem_vin)
    ein = pltpu.async_copy(Ef_hbm.at[pl.ds(ebase * 4, TE * 4)],
                           eidx_f, sem_ein)
    pltpu.sync_copy(nWf_hbm, node_f)
    pltpu.sync_copy(eWf_hbm, edge_f)

    def build_pg(pg, carry):
        pvec = pg * L + iota
        a = (pvec >> 4) << 7
        b = (pvec & 15) << 7
        w = pvec << 7

        @plsc.parallel_loop(0, D, unroll=8)
        def _cols(c):
            va = plsc.load_gather(edge_f, [a + c])
            vb = plsc.load_gather(edge_f, [b + c])
            plsc.store_scatter(w2_f, [w + c], va + vb)
        return carry
    lax.fori_loop(0, 16, build_pg, 0)

    vin.wait()
    ein.wait()

    VW = L * D

    def wait_o0(nwords):
        pltpu.make_async_copy(out_f.at[pl.ds(0, nwords)],
                              eoutf_hbm.at[pl.ds(0, nwords)], sem_o0).wait()

    def wait_o1(nwords):
        pltpu.make_async_copy(out_f.at[pl.ds(0, nwords)],
                              eoutf_hbm.at[pl.ds(0, nwords)], sem_o1).wait()

    def vchunk(g, carry):
        b = g & 1
        off = b * EWORDS
        @pl.when(g < vgn)
        def _active():
            @pl.when((g >= 2) & (b == 0))
            def _w0():
                wait_o0(VW)
            @pl.when((g >= 2) & (b == 1))
            def _w1():
                wait_o1(VW)
            ivec = (g * L + iota) << 2
            i0 = plsc.load_gather(vidx_f, [ivec]) << 7
            i1 = plsc.load_gather(vidx_f, [ivec + 1]) << 7
            i2 = plsc.load_gather(vidx_f, [ivec + 2]) << 7
            i3 = plsc.load_gather(vidx_f, [ivec + 3]) << 7
            ovec = (iota << 7) + off

            @plsc.parallel_loop(0, D, unroll=8)
            def _vcols(c):
                acc = (plsc.load_gather(node_f, [i0 + c])
                       + plsc.load_gather(node_f, [i1 + c])
                       + plsc.load_gather(node_f, [i2 + c])
                       + plsc.load_gather(node_f, [i3 + c]))
                plsc.store_scatter(out_f, [ovec + c], acc)
            dst = voutf_hbm.at[pl.ds((vbase + g * L) * D, VW)]
            src = out_f.at[pl.ds(off, VW)]
            @pl.when(b == 0)
            def _i0():
                pltpu.async_copy(src, dst, sem_o0)
            @pl.when(b == 1)
            def _i1():
                pltpu.async_copy(src, dst, sem_o1)
        return carry
    lax.fori_loop(0, VG_MAX, vchunk, 0)

    @pl.when(vgn >= 2)
    def _dv():
        wait_o0(VW)
        wait_o1(VW)

    def echunk(i, carry):
        b = i & 1
        off = b * EWORDS
        @pl.when((i >= 2) & (b == 0))
        def _w0():
            wait_o0(EWORDS)
        @pl.when((i >= 2) & (b == 1))
        def _w1():
            wait_o1(EWORDS)
        def grp(g, carry2):
            lvec = ((i * EG + g) * L + iota) << 2
            e0 = plsc.load_gather(eidx_f, [lvec])
            e1 = plsc.load_gather(eidx_f, [lvec + 1])
            e2 = plsc.load_gather(eidx_f, [lvec + 2])
            e3 = plsc.load_gather(eidx_f, [lvec + 3])
            p01 = ((e0 << 4) + e1) << 7
            p23 = ((e2 << 4) + e3) << 7
            ovec = ((g * L + iota) << 7) + off

            @plsc.parallel_loop(0, D, unroll=8)
            def _cols(c):
                acc = (plsc.load_gather(w2_f, [p01 + c])
                       + plsc.load_gather(w2_f, [p23 + c]))
                plsc.store_scatter(out_f, [ovec + c], acc)
            return carry2
        lax.fori_loop(0, EG, grp, 0)
        dst = eoutf_hbm.at[pl.ds((ebase + i * ECHUNK) * D, EWORDS)]
        src = out_f.at[pl.ds(off, EWORDS)]
        @pl.when(b == 0)
        def _i0():
            pltpu.async_copy(src, dst, sem_o0)
        @pl.when(b == 1)
        def _i1():
            pltpu.async_copy(src, dst, sem_o1)
        return carry
    lax.fori_loop(0, ENCHUNKS, echunk, 0)
    wait_o0(EWORDS)
    wait_o1(EWORDS)


@jax.jit
def _run(V, E, node_W, edge_W):
    Vf = jnp.pad(V.reshape(-1), (0, (VPAD - NV) * 4))
    Ef = E.reshape(-1)
    mesh = plsc.VectorSubcoreMesh(core_axis_name="c", subcore_axis_name="s",
                                  num_cores=NC, num_subcores=NS)
    f = pl.kernel(
        _body,
        out_type=(
            jax.ShapeDtypeStruct((NV * D,), jnp.float32),
            jax.ShapeDtypeStruct((NE * D,), jnp.float32),
        ),
        mesh=mesh,
        scratch_types=[
            pltpu.VMEM((128 * D,), jnp.float32),
            pltpu.VMEM((16 * D,), jnp.float32),
            pltpu.VMEM((256 * D,), jnp.float32),
            pltpu.VMEM((VROWS_MAX * 4,), jnp.int32),
            pltpu.VMEM((TE * 4,), jnp.int32),
            pltpu.VMEM((2 * EWORDS,), jnp.float32),
            pltpu.SemaphoreType.DMA,
            pltpu.SemaphoreType.DMA,
            pltpu.SemaphoreType.DMA,
            pltpu.SemaphoreType.DMA,
        ],
        compiler_params=pltpu.CompilerParams(needs_layout_passes=False),
    )
    voutf, eoutf = f(Vf, Ef, node_W.reshape(-1), edge_W.reshape(-1))
    return voutf.reshape(NV, D), eoutf.reshape(NE, D)


def kernel(V, E, node_W, edge_W):
    return _run(V, E, node_W, edge_W)
